# Initial kernel scaffold; baseline (speedup 1.0000x reference)
import jax
import jax.numpy as jnp
from jax.experimental import pallas as pl


def kernel(inp):
    raise NotImplementedError("write your pallas kernel here")


# speedup vs baseline: 10.4205x; 10.4205x over previous
import functools

import jax
import jax.numpy as jnp
import numpy as np
from jax import lax
from jax.experimental import pallas as pl
from jax.experimental.pallas import tpu as pltpu
from jax.experimental.pallas import tpu_sc as plsc

ROWS = 128
N = 32768
TOPK = 256
LANES = 16
NCHUNK = N // LANES
NB = 256
NGROUPS = NB // LANES
NWORKERS = 32
ROWS_PER = ROWS // NWORKERS
MINT = np.int32(-(2 ** 31))


def _keys(x):
    b = plsc.bitcast(x, jnp.int32)
    m = jnp.right_shift(b, 31)
    s = b ^ (m & jnp.int32(0x7FFFFFFF))
    s = jnp.where(b == MINT, jnp.int32(0), s)
    return s, s ^ MINT


def _sparsify_body(inp_hbm, out_hbm, row_a, row_b, cand_v, hist_v, sem_in, sem_out):
    wid = lax.axis_index("s") * 2 + lax.axis_index("c")
    lane = lax.iota(jnp.int32, LANES)
    ones = jnp.ones((LANES,), jnp.int32)
    zeros16 = jnp.zeros((LANES,), jnp.int32)

    def clear(i, _):
        hist_v[pl.ds(i * LANES, LANES)] = zeros16
        return 0

    lax.fori_loop(0, NB * LANES // LANES, clear, 0)

    def scan_hist(need):
        def g_body(gi, carry):
            found, sel_b, cnt_gt, c_in, above = carry
            g = (NGROUPS - 1) - gi
            acc = zeros16
            for l in range(LANES):
                off = l * NB + g * LANES
                acc = acc + hist_v[pl.ds(off, LANES)]
                hist_v[pl.ds(off, LANES)] = zeros16
            rev = lax.rev(acc, (0,))
            cum = jnp.cumsum(rev) + above
            mask = cum >= need
            mi = mask.astype(jnp.int32)
            first = jnp.logical_and(mask, jnp.cumsum(mi) == 1)
            any_m = jnp.sum(mi) > 0
            bvec = g * LANES + (LANES - 1) - lane
            take = jnp.logical_and(any_m, jnp.logical_not(found))
            sel_b = jnp.where(take, jnp.sum(jnp.where(first, bvec, 0)), sel_b)
            cnt_gt = jnp.where(take, jnp.sum(jnp.where(first, cum - rev, 0)), cnt_gt)
            c_in = jnp.where(take, jnp.sum(jnp.where(first, rev, 0)), c_in)
            found = jnp.logical_or(found, any_m)
            above = above + jnp.sum(acc)
            return found, sel_b, cnt_gt, c_in, above

        init = (jnp.bool_(False), jnp.int32(0), jnp.int32(0), jnp.int32(0),
                jnp.int32(0))
        _, sel_b, cnt_gt, c_in, _ = lax.fori_loop(0, NGROUPS, g_body, init)
        return sel_b, need - cnt_gt, c_in

    def process_row(row_v):
        def h1(i, _):
            _, u = _keys(row_v[pl.ds(i * LANES, LANES)])
            d = jnp.right_shift(u, 24) & 0xFF
            plsc.addupdate_scatter(hist_v, [lane * NB + d], ones)
            return 0

        lax.fori_loop(0, NCHUNK, h1, 0)
        b1, need, c = scan_hist(jnp.int32(TOPK))

        def c1_body(i, off):
            _, u = _keys(row_v[pl.ds(i * LANES, LANES)])
            m = (jnp.right_shift(u, 24) & 0xFF) == b1
            plsc.store_compressed(cand_v.at[pl.ds(off, LANES)], u, mask=m)
            return off + jnp.sum(m.astype(jnp.int32))

        lax.fori_loop(0, NCHUNK, c1_body, jnp.int32(0))
        tkey = jnp.left_shift(b1, 24)

        for shift in (16, 8, 0):
            nch = (c + LANES - 1) // LANES

            def hl(i, _, shift=shift, c=c):
                u = cand_v[pl.ds(i * LANES, LANES)]
                valid = (i * LANES + lane) < c
                d = jnp.right_shift(u, shift) & 0xFF
                plsc.addupdate_scatter(hist_v, [lane * NB + d], ones, mask=valid)
                return 0

            lax.fori_loop(0, nch, hl, 0)
            bl, need, c_next = scan_hist(need)
            tkey = tkey | jnp.left_shift(bl, shift)

            if shift != 0:
                def cl(i, off, shift=shift, c=c, bl=bl):
                    u = cand_v[pl.ds(i * LANES, LANES)]
                    valid = (i * LANES + lane) < c
                    m = jnp.logical_and(valid,
                                        (jnp.right_shift(u, shift) & 0xFF) == bl)
                    plsc.store_compressed(cand_v.at[pl.ds(off, LANES)], u, mask=m)
                    return off + jnp.sum(m.astype(jnp.int32))

                lax.fori_loop(0, nch, cl, jnp.int32(0))
            c = c_next

        ts = tkey ^ MINT
        need_eq = need

        def f_body(i, run):
            x = row_v[pl.ds(i * LANES, LANES)]
            s, _ = _keys(x)
            eq = s == ts
            ei = eq.astype(jnp.int32)
            inc = jnp.cumsum(ei)
            keep = jnp.logical_or(
                s > ts, jnp.logical_and(eq, (run + inc) <= need_eq))
            row_v[pl.ds(i * LANES, LANES)] = jnp.where(keep, x, jnp.float32(0.0))
            return run + jnp.sum(ei)

        lax.fori_loop(0, NCHUNK, f_body, jnp.int32(0))

    bufs = (row_a, row_b)
    row0 = wid * ROWS_PER
    loads = [None] * ROWS_PER
    stores = [None] * ROWS_PER
    loads[0] = pltpu.async_copy(inp_hbm.at[row0], bufs[0], sem_in)
    for j in range(ROWS_PER):
        buf = bufs[j % 2]
        loads[j].wait()
        if j + 1 < ROWS_PER:
            if j >= 1:
                stores[j - 1].wait()
            loads[j + 1] = pltpu.async_copy(
                inp_hbm.at[row0 + j + 1], bufs[(j + 1) % 2], sem_in)
        process_row(buf)
        stores[j] = pltpu.async_copy(buf, out_hbm.at[row0 + j], sem_out)
    stores[ROWS_PER - 2].wait()
    stores[ROWS_PER - 1].wait()


_sparsify = functools.partial(
    pl.kernel,
    out_type=jax.ShapeDtypeStruct((ROWS, N), jnp.float32),
    mesh=plsc.VectorSubcoreMesh(core_axis_name="c", subcore_axis_name="s",
                                num_cores=2, num_subcores=16),
    scratch_types=[
        pltpu.VMEM((N,), jnp.float32),
        pltpu.VMEM((N,), jnp.float32),
        pltpu.VMEM((N,), jnp.int32),
        pltpu.VMEM((NB * LANES,), jnp.int32),
        pltpu.SemaphoreType.DMA,
        pltpu.SemaphoreType.DMA,
    ],
    compiler_params=pltpu.CompilerParams(needs_layout_passes=False),
)(_sparsify_body)


def kernel(inp):
    return _sparsify(inp)


# speedup vs baseline: 13.7280x; 1.3174x over previous
import functools

import jax
import jax.numpy as jnp
import numpy as np
from jax import lax
from jax.experimental import pallas as pl
from jax.experimental.pallas import tpu as pltpu
from jax.experimental.pallas import tpu_sc as plsc

ROWS = 128
N = 32768
TOPK = 256
LANES = 16
NCHUNK = N // LANES
NB = 256
NGROUPS = NB // LANES
NWORKERS = 32
ROWS_PER = ROWS // NWORKERS
MINT = np.int32(-(2 ** 31))


def _keys(x):
    b = plsc.bitcast(x, jnp.int32)
    m = jnp.right_shift(b, 31)
    s = b ^ (m & jnp.int32(0x7FFFFFFF))
    s = jnp.where(b == MINT, jnp.int32(0), s)
    return s, s ^ MINT


def _sparsify_body(inp_hbm, out_hbm, row_a, row_b, cand_v, hist_v, sem_in, sem_out):
    wid = lax.axis_index("s") * 2 + lax.axis_index("c")
    lane = lax.iota(jnp.int32, LANES)
    lane_nb = lane * NB
    ones = jnp.ones((LANES,), jnp.int32)
    zeros16 = jnp.zeros((LANES,), jnp.int32)

    def clear(i, _):
        hist_v[pl.ds(i * LANES, LANES)] = zeros16
        return 0

    lax.fori_loop(0, NB * LANES // LANES, clear, 0)

    def scan_hist(need):
        def g_body(gi, carry):
            sel_b, cnt_gt, c_in, above = carry
            g = (NGROUPS - 1) - gi
            acc = zeros16
            for l in range(LANES):
                off = l * NB + g * LANES
                acc = acc + hist_v[pl.ds(off, LANES)]
                hist_v[pl.ds(off, LANES)] = zeros16
            gsum = jnp.sum(acc)
            take = jnp.logical_and(above < need, above + gsum >= need)

            def fine(_):
                rev = lax.rev(acc, (0,))
                cum = jnp.cumsum(rev) + above
                mask = cum >= need
                first = jnp.logical_and(mask,
                                        jnp.cumsum(mask.astype(jnp.int32)) == 1)
                bvec = g * LANES + (LANES - 1) - lane
                nb = jnp.sum(jnp.where(first, bvec, 0))
                ng = jnp.sum(jnp.where(first, cum - rev, 0))
                nc = jnp.sum(jnp.where(first, rev, 0))
                return nb, ng, nc

            sel_b, cnt_gt, c_in = lax.cond(
                take, fine, lambda _: (sel_b, cnt_gt, c_in), 0)
            return sel_b, cnt_gt, c_in, above + gsum

        init = (jnp.int32(0), jnp.int32(0), jnp.int32(0), jnp.int32(0))
        sel_b, cnt_gt, c_in, _ = lax.fori_loop(0, NGROUPS, g_body, init)
        return sel_b, need - cnt_gt, c_in

    def process_row(row_v):
        def h1(i, _):
            _, u = _keys(row_v[pl.ds(i * LANES, LANES)])
            d = jnp.right_shift(u, 24) & 0xFF
            plsc.addupdate_scatter(hist_v, [lane_nb + d], ones)
            return 0

        lax.fori_loop(0, NCHUNK, h1, 0, unroll=4)
        b1, need, c = scan_hist(jnp.int32(TOPK))

        def c1_body(i, off):
            s, u = _keys(row_v[pl.ds(i * LANES, LANES)])
            m = (jnp.right_shift(u, 24) & 0xFF) == b1
            plsc.store_compressed(cand_v.at[pl.ds(off, LANES)], s, mask=m)
            return off + plsc.all_reduce_population_count(m)[0]

        lax.fori_loop(0, NCHUNK, c1_body, jnp.int32(0), unroll=4)
        tkey = jnp.left_shift(b1, 24)

        for shift in (16, 8, 0):
            nch = (c + LANES - 1) // LANES

            def hl(i, _, shift=shift, c=c):
                s = cand_v[pl.ds(i * LANES, LANES)]
                valid = (i * LANES + lane) < c
                d = jnp.right_shift(s, shift) & 0xFF
                plsc.addupdate_scatter(hist_v, [lane_nb + d], ones, mask=valid)
                return 0

            lax.fori_loop(0, nch, hl, 0)
            bl, need, c_next = scan_hist(need)
            tkey = tkey | jnp.left_shift(bl, shift)

            if shift != 0:
                def cl(i, off, shift=shift, c=c, bl=bl):
                    s = cand_v[pl.ds(i * LANES, LANES)]
                    valid = (i * LANES + lane) < c
                    m = jnp.logical_and(valid,
                                        (jnp.right_shift(s, shift) & 0xFF) == bl)
                    plsc.store_compressed(cand_v.at[pl.ds(off, LANES)], s, mask=m)
                    return off + plsc.all_reduce_population_count(m)[0]

                lax.fori_loop(0, nch, cl, jnp.int32(0))
            c = c_next

        ts = tkey ^ MINT
        need_eq = need

        @pl.when(need_eq == c)
        def _fast():
            def f_body(i, _):
                x = row_v[pl.ds(i * LANES, LANES)]
                s, _ = _keys(x)
                row_v[pl.ds(i * LANES, LANES)] = jnp.where(
                    s >= ts, x, jnp.float32(0.0))
                return 0

            lax.fori_loop(0, NCHUNK, f_body, 0, unroll=4)

        @pl.when(need_eq != c)
        def _tie():
            def f_body(i, run):
                x = row_v[pl.ds(i * LANES, LANES)]
                s, _ = _keys(x)
                eq = s == ts
                inc = jnp.cumsum(eq.astype(jnp.int32))
                keep = jnp.logical_or(
                    s > ts, jnp.logical_and(eq, (run + inc) <= need_eq))
                row_v[pl.ds(i * LANES, LANES)] = jnp.where(
                    keep, x, jnp.float32(0.0))
                return run + plsc.all_reduce_population_count(eq)[0]

            lax.fori_loop(0, NCHUNK, f_body, jnp.int32(0))

    bufs = (row_a, row_b)
    row0 = wid * ROWS_PER
    loads = [None] * ROWS_PER
    stores = [None] * ROWS_PER
    loads[0] = pltpu.async_copy(inp_hbm.at[row0], bufs[0], sem_in)
    for j in range(ROWS_PER):
        buf = bufs[j % 2]
        loads[j].wait()
        if j + 1 < ROWS_PER:
            if j >= 1:
                stores[j - 1].wait()
            loads[j + 1] = pltpu.async_copy(
                inp_hbm.at[row0 + j + 1], bufs[(j + 1) % 2], sem_in)
        process_row(buf)
        stores[j] = pltpu.async_copy(buf, out_hbm.at[row0 + j], sem_out)
    stores[ROWS_PER - 2].wait()
    stores[ROWS_PER - 1].wait()


_sparsify = functools.partial(
    pl.kernel,
    out_type=jax.ShapeDtypeStruct((ROWS, N), jnp.float32),
    mesh=plsc.VectorSubcoreMesh(core_axis_name="c", subcore_axis_name="s",
                                num_cores=2, num_subcores=16),
    scratch_types=[
        pltpu.VMEM((N,), jnp.float32),
        pltpu.VMEM((N,), jnp.float32),
        pltpu.VMEM((N,), jnp.int32),
        pltpu.VMEM((NB * LANES,), jnp.int32),
        pltpu.SemaphoreType.DMA,
        pltpu.SemaphoreType.DMA,
    ],
    compiler_params=pltpu.CompilerParams(needs_layout_passes=False),
)(_sparsify_body)


def kernel(inp):
    return _sparsify(inp)
